# Initial kernel scaffold; baseline (speedup 1.0000x reference)
import jax
import jax.numpy as jnp
from jax.experimental import pallas as pl


def kernel(dense_features, sparse_indices, emb_table, W1, b1, W2, b2, W3, b3, Wo, bo):
    raise NotImplementedError("write your pallas kernel here")


# speedup vs baseline: 4.4053x; 4.4053x over previous
import functools

import jax
import jax.numpy as jnp
from jax import lax
from jax.experimental import pallas as pl
from jax.experimental.pallas import tpu as pltpu
from jax.experimental.pallas import tpu_sc as plsc

_FPAD = 32


def _proj_kernel(a_ref, emb_ref, out_ref):
    out_ref[...] = lax.dot_general(
        a_ref[...], emb_ref[...],
        dimension_numbers=(((1,), (1,)), ((), ())),
        preferred_element_type=jnp.float32)


def _transpose_kernel(idx_ref, out_ref):
    x = idx_ref[...]
    b, f = x.shape
    xp = jnp.concatenate([x, jnp.zeros((b, _FPAD - f), jnp.int32)], axis=1)
    out_ref[...] = xp.T


def _make_sc_gather(Vp, B):
    mesh = plsc.VectorSubcoreMesh(core_axis_name="c", subcore_axis_name="s")

    @functools.partial(
        pl.kernel,
        out_type=jax.ShapeDtypeStruct((_FPAD, B), jnp.float32),
        mesh=mesh,
        compiler_params=pltpu.CompilerParams(
            needs_layout_passes=False, use_tc_tiling_on_sc=False),
        scratch_types=[
            pltpu.VMEM((Vp,), jnp.float32),
            pltpu.VMEM((B,), jnp.int32),
            pltpu.VMEM((B,), jnp.float32),
        ],
    )
    def sc_gather(projT_hbm, idxT_hbm, out_hbm, tab_v, idx_v, out_v):
        f = lax.axis_index("s") * 2 + lax.axis_index("c")
        pltpu.sync_copy(projT_hbm.at[f], tab_v)
        pltpu.sync_copy(idxT_hbm.at[f], idx_v)

        def body(i, carry):
            ids = idx_v[pl.ds(i * 16, 16)]
            out_v[pl.ds(i * 16, 16)] = plsc.load_gather(tab_v, [ids])
            return carry

        lax.fori_loop(0, B // 16, body, 0)
        pltpu.sync_copy(out_v, out_hbm.at[f])

    return sc_gather


def _dense_kernel(x_ref, part_ref, w1_ref, b1_ref, w2_ref, b2_ref,
                  w3_ref, b3_ref, woh_ref, bo_ref, out_ref):
    h = jnp.maximum(
        jnp.dot(x_ref[...], w1_ref[...],
                preferred_element_type=jnp.float32) + b1_ref[...], 0.0)
    h = jnp.maximum(
        jnp.dot(h, w2_ref[...],
                preferred_element_type=jnp.float32) + b2_ref[...], 0.0)
    h = jnp.maximum(
        jnp.dot(h, w3_ref[...],
                preferred_element_type=jnp.float32) + b3_ref[...], 0.0)
    dense = jnp.dot(h, woh_ref[...], preferred_element_type=jnp.float32)
    sparse = jnp.sum(part_ref[...], axis=0)[:, None]
    out_ref[...] = dense + sparse + bo_ref[0, 0]


def kernel(dense_features, sparse_indices, emb_table, W1, b1, W2, b2, W3, b3,
           Wo, bo):
    B, DIN = dense_features.shape
    _, F = sparse_indices.shape
    V, D = emb_table.shape

    VC = 2048
    G = -(-V // VC)
    Vp = G * VC

    A = Wo[:F * D, 0].reshape(F, D)
    A_pad = jnp.zeros((_FPAD, D), jnp.float32).at[:F].set(A)
    woh = Wo[F * D:, :]

    projT = pl.pallas_call(
        _proj_kernel,
        grid=(G,),
        in_specs=[pl.BlockSpec((_FPAD, D), lambda i: (0, 0)),
                  pl.BlockSpec((VC, D), lambda i: (i, 0))],
        out_specs=pl.BlockSpec((_FPAD, VC), lambda i: (0, i)),
        out_shape=jax.ShapeDtypeStruct((_FPAD, Vp), jnp.float32),
    )(A_pad, emb_table)

    idxT = pl.pallas_call(
        _transpose_kernel,
        out_shape=jax.ShapeDtypeStruct((_FPAD, B), jnp.int32),
    )(sparse_indices)

    partials = _make_sc_gather(Vp, B)(projT, idxT)

    blk = 1024
    out = pl.pallas_call(
        _dense_kernel,
        grid=(B // blk,),
        in_specs=[pl.BlockSpec((blk, DIN), lambda i: (i, 0)),
                  pl.BlockSpec((_FPAD, blk), lambda i: (0, i)),
                  pl.BlockSpec((DIN, 512), lambda i: (0, 0)),
                  pl.BlockSpec((1, 512), lambda i: (0, 0)),
                  pl.BlockSpec((512, 256), lambda i: (0, 0)),
                  pl.BlockSpec((1, 256), lambda i: (0, 0)),
                  pl.BlockSpec((256, D), lambda i: (0, 0)),
                  pl.BlockSpec((1, D), lambda i: (0, 0)),
                  pl.BlockSpec((D, 1), lambda i: (0, 0)),
                  pl.BlockSpec((1, 1), lambda i: (0, 0))],
        out_specs=pl.BlockSpec((blk, 1), lambda i: (i, 0)),
        out_shape=jax.ShapeDtypeStruct((B, 1), jnp.float32),
    )(dense_features, partials, W1, b1.reshape(1, -1), W2, b2.reshape(1, -1),
      W3, b3.reshape(1, -1), woh, bo.reshape(1, 1))
    return out


# speedup vs baseline: 4.9073x; 1.1139x over previous
import functools

import jax
import jax.numpy as jnp
from jax import lax
from jax.experimental import pallas as pl
from jax.experimental.pallas import tpu as pltpu
from jax.experimental.pallas import tpu_sc as plsc

_FPAD = 32


def _proj_kernel(a_ref, emb_ref, out_ref):
    out_ref[...] = lax.dot_general(
        a_ref[...], emb_ref[...],
        dimension_numbers=(((1,), (1,)), ((), ())),
        preferred_element_type=jnp.float32)


def _transpose_kernel(idx_ref, out_ref):
    x = idx_ref[...]
    b, f = x.shape
    xp = jnp.concatenate([x, jnp.zeros((b, _FPAD - f), jnp.int32)], axis=1)
    out_ref[...] = xp.T


def _make_sc_gather(Vp, B):
    mesh = plsc.VectorSubcoreMesh(core_axis_name="c", subcore_axis_name="s")

    @functools.partial(
        pl.kernel,
        out_type=jax.ShapeDtypeStruct((_FPAD, B), jnp.float32),
        mesh=mesh,
        compiler_params=pltpu.CompilerParams(needs_layout_passes=False),
        scratch_types=[
            pltpu.VMEM((Vp,), jnp.float32),
            pltpu.VMEM((B,), jnp.int32),
            pltpu.VMEM((B,), jnp.float32),
        ],
    )
    def sc_gather(projT_hbm, idxT_hbm, out_hbm, tab_v, idx_v, out_v):
        f = lax.axis_index("s") * 2 + lax.axis_index("c")
        pltpu.sync_copy(projT_hbm.at[f], tab_v)
        pltpu.sync_copy(idxT_hbm.at[f], idx_v)

        def body(i, carry):
            ids = idx_v[pl.ds(i * 16, 16)]
            out_v[pl.ds(i * 16, 16)] = plsc.load_gather(tab_v, [ids])
            return carry

        lax.fori_loop(0, B // 16, body, 0)
        pltpu.sync_copy(out_v, out_hbm.at[f])

    return sc_gather


def _dense_kernel(x_ref, part_ref, w1_ref, b1_ref, w2_ref, b2_ref,
                  w3_ref, b3_ref, woh_ref, bo_ref, out_ref):
    h = jnp.maximum(
        jnp.dot(x_ref[...], w1_ref[...],
                preferred_element_type=jnp.float32) + b1_ref[...], 0.0)
    h = jnp.maximum(
        jnp.dot(h, w2_ref[...],
                preferred_element_type=jnp.float32) + b2_ref[...], 0.0)
    h = jnp.maximum(
        jnp.dot(h, w3_ref[...],
                preferred_element_type=jnp.float32) + b3_ref[...], 0.0)
    dense = jnp.dot(h, woh_ref[...], preferred_element_type=jnp.float32)
    sparse = jnp.sum(part_ref[...], axis=0)[:, None]
    out_ref[...] = dense + sparse + bo_ref[0, 0]


def kernel(dense_features, sparse_indices, emb_table, W1, b1, W2, b2, W3, b3,
           Wo, bo):
    B, DIN = dense_features.shape
    _, F = sparse_indices.shape
    V, D = emb_table.shape

    VC = 2048
    G = -(-V // VC)
    Vp = G * VC

    A = Wo[:F * D, 0].reshape(F, D)
    A_pad = jnp.zeros((_FPAD, D), jnp.float32).at[:F].set(A)
    woh = Wo[F * D:, :]

    projT = pl.pallas_call(
        _proj_kernel,
        grid=(G,),
        in_specs=[pl.BlockSpec((_FPAD, D), lambda i: (0, 0)),
                  pl.BlockSpec((VC, D), lambda i: (i, 0))],
        out_specs=pl.BlockSpec((_FPAD, VC), lambda i: (0, i)),
        out_shape=jax.ShapeDtypeStruct((_FPAD, Vp), jnp.float32),
    )(A_pad, emb_table)

    idxT = pl.pallas_call(
        _transpose_kernel,
        out_shape=jax.ShapeDtypeStruct((_FPAD, B), jnp.int32),
    )(sparse_indices)

    partials = _make_sc_gather(Vp, B)(projT, idxT)

    blk = 1024
    out = pl.pallas_call(
        _dense_kernel,
        grid=(B // blk,),
        in_specs=[pl.BlockSpec((blk, DIN), lambda i: (i, 0)),
                  pl.BlockSpec((_FPAD, blk), lambda i: (0, i)),
                  pl.BlockSpec((DIN, 512), lambda i: (0, 0)),
                  pl.BlockSpec((1, 512), lambda i: (0, 0)),
                  pl.BlockSpec((512, 256), lambda i: (0, 0)),
                  pl.BlockSpec((1, 256), lambda i: (0, 0)),
                  pl.BlockSpec((256, D), lambda i: (0, 0)),
                  pl.BlockSpec((1, D), lambda i: (0, 0)),
                  pl.BlockSpec((D, 1), lambda i: (0, 0)),
                  pl.BlockSpec((1, 1), lambda i: (0, 0))],
        out_specs=pl.BlockSpec((blk, 1), lambda i: (i, 0)),
        out_shape=jax.ShapeDtypeStruct((B, 1), jnp.float32),
    )(dense_features, partials, W1, b1.reshape(1, -1), W2, b2.reshape(1, -1),
      W3, b3.reshape(1, -1), woh, bo.reshape(1, 1))
    return out


# speedup vs baseline: 5.9377x; 1.2100x over previous
import functools

import jax
import jax.numpy as jnp
from jax import lax
from jax.experimental import pallas as pl
from jax.experimental.pallas import tpu as pltpu
from jax.experimental.pallas import tpu_sc as plsc

_FPAD = 32


def _proj_kernel(a_ref, emb_ref, out_ref):
    out_ref[...] = lax.dot_general(
        a_ref[...], emb_ref[...],
        dimension_numbers=(((1,), (1,)), ((), ())),
        preferred_element_type=jnp.float32)


def _transpose_kernel(idx_ref, out_ref):
    out_ref[...] = idx_ref[...].T


def _make_sc_gather(Vp, B, F):
    mesh = plsc.VectorSubcoreMesh(core_axis_name="c", subcore_axis_name="s")

    @functools.partial(
        pl.kernel,
        out_type=jax.ShapeDtypeStruct((F, B), jnp.float32),
        mesh=mesh,
        compiler_params=pltpu.CompilerParams(needs_layout_passes=False),
        scratch_types=[
            pltpu.VMEM((Vp,), jnp.float32),
            pltpu.VMEM((B,), jnp.int32),
            pltpu.VMEM((B,), jnp.float32),
        ],
    )
    def sc_gather(projT_hbm, idxT_hbm, out_hbm, tab_v, idx_v, out_v):
        f = lax.axis_index("s") * 2 + lax.axis_index("c")

        @pl.when(f < F)
        def _():
            pltpu.sync_copy(projT_hbm.at[f], tab_v)
            pltpu.sync_copy(idxT_hbm.at[f], idx_v)

            def body(i, carry):
                ids = idx_v[pl.ds(i * 16, 16)]
                out_v[pl.ds(i * 16, 16)] = plsc.load_gather(tab_v, [ids])
                return carry

            lax.fori_loop(0, B // 16, body, 0)
            pltpu.sync_copy(out_v, out_hbm.at[f])

    return sc_gather


def _dense_kernel(x_ref, part_ref, w1_ref, b1_ref, w2_ref, b2_ref,
                  w3_ref, b3_ref, woh_ref, bo_ref, out_ref):
    h = jnp.maximum(
        jnp.dot(x_ref[...], w1_ref[...],
                preferred_element_type=jnp.float32) + b1_ref[...], 0.0)
    h = jnp.maximum(
        jnp.dot(h, w2_ref[...],
                preferred_element_type=jnp.float32) + b2_ref[...], 0.0)
    h = jnp.maximum(
        jnp.dot(h, w3_ref[...],
                preferred_element_type=jnp.float32) + b3_ref[...], 0.0)
    dense = jnp.dot(h, woh_ref[...], preferred_element_type=jnp.float32)
    sparse = jnp.sum(part_ref[...], axis=0)[:, None]
    out_ref[...] = dense + sparse + bo_ref[0, 0]


def kernel(dense_features, sparse_indices, emb_table, W1, b1, W2, b2, W3, b3,
           Wo, bo):
    B, DIN = dense_features.shape
    _, F = sparse_indices.shape
    V, D = emb_table.shape

    VC = 8192
    G = -(-V // VC)
    Vp = G * VC

    A = Wo[:F * D, 0].reshape(F, D)
    woh = Wo[F * D:, :]

    projT = pl.pallas_call(
        _proj_kernel,
        grid=(G,),
        in_specs=[pl.BlockSpec((F, D), lambda i: (0, 0)),
                  pl.BlockSpec((VC, D), lambda i: (i, 0))],
        out_specs=pl.BlockSpec((F, VC), lambda i: (0, i)),
        out_shape=jax.ShapeDtypeStruct((F, Vp), jnp.float32),
    )(A, emb_table)

    idxT = pl.pallas_call(
        _transpose_kernel,
        out_shape=jax.ShapeDtypeStruct((F, B), jnp.int32),
    )(sparse_indices)

    partials = _make_sc_gather(Vp, B, F)(projT, idxT)

    blk = 1024
    out = pl.pallas_call(
        _dense_kernel,
        grid=(B // blk,),
        in_specs=[pl.BlockSpec((blk, DIN), lambda i: (i, 0)),
                  pl.BlockSpec((F, blk), lambda i: (0, i)),
                  pl.BlockSpec((DIN, 512), lambda i: (0, 0)),
                  pl.BlockSpec((1, 512), lambda i: (0, 0)),
                  pl.BlockSpec((512, 256), lambda i: (0, 0)),
                  pl.BlockSpec((1, 256), lambda i: (0, 0)),
                  pl.BlockSpec((256, D), lambda i: (0, 0)),
                  pl.BlockSpec((1, D), lambda i: (0, 0)),
                  pl.BlockSpec((D, 1), lambda i: (0, 0)),
                  pl.BlockSpec((1, 1), lambda i: (0, 0))],
        out_specs=pl.BlockSpec((blk, 1), lambda i: (i, 0)),
        out_shape=jax.ShapeDtypeStruct((B, 1), jnp.float32),
    )(dense_features, partials, W1, b1.reshape(1, -1), W2, b2.reshape(1, -1),
      W3, b3.reshape(1, -1), woh, bo.reshape(1, 1))
    return out


# speedup vs baseline: 11.0360x; 1.8586x over previous
import functools

import jax
import jax.numpy as jnp
from jax import lax
from jax.experimental import pallas as pl
from jax.experimental.pallas import tpu as pltpu
from jax.experimental.pallas import tpu_sc as plsc


def _proj_kernel(a_ref, embt_ref, out_ref):
    out_ref[...] = jnp.dot(a_ref[...], embt_ref[...],
                           preferred_element_type=jnp.float32)


def _make_sc_gather(Vp, B, F):
    mesh = plsc.VectorSubcoreMesh(core_axis_name="c", subcore_axis_name="s")

    @functools.partial(
        pl.kernel,
        out_type=jax.ShapeDtypeStruct((F, B), jnp.float32),
        mesh=mesh,
        compiler_params=pltpu.CompilerParams(needs_layout_passes=False),
        scratch_types=[
            pltpu.VMEM((Vp,), jnp.float32),
            pltpu.VMEM((B,), jnp.int32),
            pltpu.VMEM((B,), jnp.float32),
        ],
    )
    def sc_gather(projT_hbm, idxT_hbm, out_hbm, tab_v, idx_v, out_v):
        f = lax.axis_index("s") * 2 + lax.axis_index("c")

        @pl.when(f < F)
        def _():
            pltpu.sync_copy(projT_hbm.at[f], tab_v)
            pltpu.sync_copy(idxT_hbm.at[f], idx_v)

            def body(i, carry):
                ids = idx_v[pl.ds(i * 16, 16)]
                out_v[pl.ds(i * 16, 16)] = plsc.load_gather(tab_v, [ids])
                return carry

            lax.fori_loop(0, B // 16, body, 0)
            pltpu.sync_copy(out_v, out_hbm.at[f])

    return sc_gather


def _dense_kernel(xt_ref, part_ref, w1_ref, b1_ref, w2_ref, b2_ref,
                  w3_ref, b3_ref, woh_ref, bo_ref, out_ref):
    h = jnp.maximum(
        lax.dot_general(xt_ref[...], w1_ref[...],
                        dimension_numbers=(((0,), (0,)), ((), ())),
                        preferred_element_type=jnp.float32) + b1_ref[...], 0.0)
    h = jnp.maximum(
        jnp.dot(h, w2_ref[...],
                preferred_element_type=jnp.float32) + b2_ref[...], 0.0)
    h = jnp.maximum(
        jnp.dot(h, w3_ref[...],
                preferred_element_type=jnp.float32) + b3_ref[...], 0.0)
    dense = jnp.dot(h, woh_ref[...], preferred_element_type=jnp.float32)
    sparse = jnp.sum(part_ref[...], axis=0)[:, None]
    out_ref[...] = dense + sparse + bo_ref[0, 0]


def kernel(dense_features, sparse_indices, emb_table, W1, b1, W2, b2, W3, b3,
           Wo, bo):
    B, DIN = dense_features.shape
    _, F = sparse_indices.shape
    V, D = emb_table.shape

    VC = 8192
    G = -(-V // VC)
    Vp = G * VC

    embT = emb_table.T
    idxT = sparse_indices.T
    xT = dense_features.T
    A = Wo[:F * D, 0].reshape(F, D)
    woh = Wo[F * D:, :]

    projT = pl.pallas_call(
        _proj_kernel,
        grid=(G,),
        in_specs=[pl.BlockSpec((F, D), lambda i: (0, 0)),
                  pl.BlockSpec((D, VC), lambda i: (0, i))],
        out_specs=pl.BlockSpec((F, VC), lambda i: (0, i)),
        out_shape=jax.ShapeDtypeStruct((F, Vp), jnp.float32),
    )(A, embT)

    partials = _make_sc_gather(Vp, B, F)(projT, idxT)

    blk = 1024
    out = pl.pallas_call(
        _dense_kernel,
        grid=(B // blk,),
        in_specs=[pl.BlockSpec((DIN, blk), lambda i: (0, i)),
                  pl.BlockSpec((F, blk), lambda i: (0, i)),
                  pl.BlockSpec((DIN, 512), lambda i: (0, 0)),
                  pl.BlockSpec((1, 512), lambda i: (0, 0)),
                  pl.BlockSpec((512, 256), lambda i: (0, 0)),
                  pl.BlockSpec((1, 256), lambda i: (0, 0)),
                  pl.BlockSpec((256, D), lambda i: (0, 0)),
                  pl.BlockSpec((1, D), lambda i: (0, 0)),
                  pl.BlockSpec((D, 1), lambda i: (0, 0)),
                  pl.BlockSpec((1, 1), lambda i: (0, 0))],
        out_specs=pl.BlockSpec((blk, 1), lambda i: (i, 0)),
        out_shape=jax.ShapeDtypeStruct((B, 1), jnp.float32),
    )(xT, partials, W1, b1.reshape(1, -1), W2, b2.reshape(1, -1),
      W3, b3.reshape(1, -1), woh, bo.reshape(1, 1))
    return out


# speedup vs baseline: 11.2822x; 1.0223x over previous
import functools

import jax
import jax.numpy as jnp
from jax import lax
from jax.experimental import pallas as pl
from jax.experimental.pallas import tpu as pltpu
from jax.experimental.pallas import tpu_sc as plsc


def _proj_kernel(a_ref, embt_ref, out_ref):
    out_ref[...] = jnp.dot(a_ref[...], embt_ref[...],
                           preferred_element_type=jnp.float32)


def _make_sc_gather(Vp, B, F):
    mesh = plsc.VectorSubcoreMesh(core_axis_name="c", subcore_axis_name="s")

    @functools.partial(
        pl.kernel,
        out_type=jax.ShapeDtypeStruct((F, B), jnp.float32),
        mesh=mesh,
        compiler_params=pltpu.CompilerParams(needs_layout_passes=False),
        scratch_types=[
            pltpu.VMEM((Vp,), jnp.float32),
            pltpu.VMEM((B,), jnp.int32),
            pltpu.VMEM((B,), jnp.float32),
        ],
    )
    def sc_gather(projT_hbm, idxT_hbm, out_hbm, tab_v, idx_v, out_v):
        f = lax.axis_index("s") * 2 + lax.axis_index("c")

        @pl.when(f < F)
        def _():
            pltpu.sync_copy(projT_hbm.at[f], tab_v)
            pltpu.sync_copy(idxT_hbm.at[f], idx_v)

            def body(i, carry):
                ids = idx_v[pl.ds(i * 16, 16)]
                out_v[pl.ds(i * 16, 16)] = plsc.load_gather(tab_v, [ids])
                return carry

            lax.fori_loop(0, B // 16, body, 0)
            pltpu.sync_copy(out_v, out_hbm.at[f])

    return sc_gather


def _dense_kernel(xt_ref, w1_ref, b1_ref, w2_ref, b2_ref,
                  w3_ref, b3_ref, woh_ref, bo_ref, out_ref):
    h = jnp.maximum(
        lax.dot_general(xt_ref[...], w1_ref[...],
                        dimension_numbers=(((0,), (0,)), ((), ())),
                        preferred_element_type=jnp.float32) + b1_ref[...], 0.0)
    h = jnp.maximum(
        jnp.dot(h, w2_ref[...],
                preferred_element_type=jnp.float32) + b2_ref[...], 0.0)
    h = jnp.maximum(
        jnp.dot(h, w3_ref[...],
                preferred_element_type=jnp.float32) + b3_ref[...], 0.0)
    out_ref[...] = (jnp.dot(h, woh_ref[...], preferred_element_type=jnp.float32)
                    + bo_ref[0, 0])


def _combine_kernel(part_ref, dense_ref, out_ref):
    out_ref[...] = jnp.sum(part_ref[...], axis=0)[:, None] + dense_ref[...]


def kernel(dense_features, sparse_indices, emb_table, W1, b1, W2, b2, W3, b3,
           Wo, bo):
    B, DIN = dense_features.shape
    _, F = sparse_indices.shape
    V, D = emb_table.shape

    VC = 8192
    G = -(-V // VC)
    Vp = G * VC

    embT = emb_table.T
    idxT = sparse_indices.T
    xT = dense_features.T
    A = Wo[:F * D, 0].reshape(F, D)
    woh = Wo[F * D:, :]

    projT = pl.pallas_call(
        _proj_kernel,
        grid=(G,),
        in_specs=[pl.BlockSpec((F, D), lambda i: (0, 0)),
                  pl.BlockSpec((D, VC), lambda i: (0, i))],
        out_specs=pl.BlockSpec((F, VC), lambda i: (0, i)),
        out_shape=jax.ShapeDtypeStruct((F, Vp), jnp.float32),
    )(A, embT)

    partials = _make_sc_gather(Vp, B, F)(projT, idxT)

    blk = 1024
    densepart = pl.pallas_call(
        _dense_kernel,
        grid=(B // blk,),
        in_specs=[pl.BlockSpec((DIN, blk), lambda i: (0, i)),
                  pl.BlockSpec((DIN, 512), lambda i: (0, 0)),
                  pl.BlockSpec((1, 512), lambda i: (0, 0)),
                  pl.BlockSpec((512, 256), lambda i: (0, 0)),
                  pl.BlockSpec((1, 256), lambda i: (0, 0)),
                  pl.BlockSpec((256, D), lambda i: (0, 0)),
                  pl.BlockSpec((1, D), lambda i: (0, 0)),
                  pl.BlockSpec((D, 1), lambda i: (0, 0)),
                  pl.BlockSpec((1, 1), lambda i: (0, 0))],
        out_specs=pl.BlockSpec((blk, 1), lambda i: (i, 0)),
        out_shape=jax.ShapeDtypeStruct((B, 1), jnp.float32),
    )(xT, W1, b1.reshape(1, -1), W2, b2.reshape(1, -1),
      W3, b3.reshape(1, -1), woh, bo.reshape(1, 1))

    out = pl.pallas_call(
        _combine_kernel,
        grid=(B // blk,),
        in_specs=[pl.BlockSpec((F, blk), lambda i: (0, i)),
                  pl.BlockSpec((blk, 1), lambda i: (i, 0))],
        out_specs=pl.BlockSpec((blk, 1), lambda i: (i, 0)),
        out_shape=jax.ShapeDtypeStruct((B, 1), jnp.float32),
    )(partials, densepart)
    return out
